# Initial kernel scaffold; baseline (speedup 1.0000x reference)
import jax
import jax.numpy as jnp
from jax.experimental import pallas as pl


def kernel(y_pred, y_true, thresholds):
    raise NotImplementedError("write your pallas kernel here")


# speedup vs baseline: 813.5274x; 813.5274x over previous
import jax
import jax.numpy as jnp
from jax.experimental import pallas as pl
from jax.experimental.pallas import tpu as pltpu

_T = 10
_EPS = 1e-7
_LANES = 1024
_ROWS = 6144
_BLK_R = 512
_NB = _ROWS // _BLK_R
_K = 1024.0


def _count_kernel(th_ref, yp_ref, yt_ref, out_ref):
    i = pl.program_id(1)

    @pl.when(i == 0)
    def _init():
        out_ref[...] = jnp.zeros_like(out_ref)

    s = yp_ref[0]
    y = yt_ref[0]
    m_pos = y > 0
    w = jnp.where(m_pos, 1.0 + _K, 1.0)
    pos_f = jnp.where(m_pos, 1.0, 0.0)

    out_ref[0, 20:21, :] += jnp.sum(pos_f, axis=0, keepdims=True)

    for j in range(_T):
        t = th_ref[j]
        v = jnp.sum(jnp.where(s > t, w, 0.0), axis=0, keepdims=True)
        tp = jnp.floor(v * (1.0 / _K))
        pp = v - _K * tp
        out_ref[0, j:j + 1, :] += pp
        out_ref[0, 10 + j:11 + j, :] += tp


def kernel(y_pred, y_true, thresholds):
    yp4 = y_pred.reshape(4, _ROWS, _LANES)
    yt3 = y_true.reshape(2, _ROWS, _LANES)

    out = pl.pallas_call(
        _count_kernel,
        grid=(2, _NB),
        in_specs=[
            pl.BlockSpec(memory_space=pltpu.SMEM),
            pl.BlockSpec((1, _BLK_R, _LANES), lambda b, i: (2 * b + 1, i, 0)),
            pl.BlockSpec((1, _BLK_R, _LANES), lambda b, i: (b, i, 0)),
        ],
        out_specs=pl.BlockSpec((1, 24, _LANES), lambda b, i: (b, 0, 0)),
        out_shape=jax.ShapeDtypeStruct((2, 24, _LANES), jnp.float32),
        compiler_params=pltpu.CompilerParams(
            dimension_semantics=("parallel", "arbitrary")),
    )(thresholds, yp4, yt3)

    pp = out[:, 0:10, :].sum(axis=(0, 2))
    tp = out[:, 10:20, :].sum(axis=(0, 2))
    npos = out[:, 20, :].sum()
    precisions = (tp + _EPS) / (pp + _EPS)
    recalls = (tp + _EPS) / (npos + _EPS)
    p = jnp.concatenate([jnp.zeros((1,), jnp.float32), precisions,
                         jnp.ones((1,), jnp.float32)])
    r = jnp.concatenate([jnp.ones((1,), jnp.float32), recalls,
                         jnp.zeros((1,), jnp.float32)])
    area = 0.5 * jnp.sum((r[1:] - r[:-1]) * (p[1:] + p[:-1]))
    return jnp.abs(area).astype(jnp.float32)


# speedup vs baseline: 2596.8145x; 3.1920x over previous
import jax
import jax.numpy as jnp
from jax.experimental import pallas as pl
from jax.experimental.pallas import tpu as pltpu

_T = 10
_EPS = 1e-7
_D = 96
_BLK_D = 8
_NB = _D // _BLK_D
_K = 4096.0


def _count_kernel(th_ref, yp_ref, yt_ref, out_ref):
    i = pl.program_id(1)

    @pl.when(i == 0)
    def _init():
        out_ref[...] = jnp.zeros_like(out_ref)

    s = yp_ref[0, 0]
    y = yt_ref[0]
    m_pos = y > 0
    w = jnp.where(m_pos, 1.0 + _K, 1.0)
    pos_f = jnp.where(m_pos, 1.0, 0.0)

    out_ref[0, 20:21, :] += jnp.sum(pos_f, axis=(0, 1))[None, :]

    for j in range(_T):
        t = th_ref[j]
        v = jnp.sum(jnp.where(s > t, w, 0.0), axis=(0, 1))
        tp = jnp.floor(v * (1.0 / _K))
        pp = v - _K * tp
        out_ref[0, j:j + 1, :] += pp[None, :]
        out_ref[0, 10 + j:11 + j, :] += tp[None, :]


def kernel(y_pred, y_true, thresholds):
    out = pl.pallas_call(
        _count_kernel,
        grid=(2, _NB),
        in_specs=[
            pl.BlockSpec(memory_space=pltpu.SMEM),
            pl.BlockSpec((1, 1, _BLK_D, 256, 256),
                         lambda b, i: (b, 1, i, 0, 0)),
            pl.BlockSpec((1, _BLK_D, 256, 256),
                         lambda b, i: (b, i, 0, 0)),
        ],
        out_specs=pl.BlockSpec((1, 24, 256), lambda b, i: (b, 0, 0)),
        out_shape=jax.ShapeDtypeStruct((2, 24, 256), jnp.float32),
        compiler_params=pltpu.CompilerParams(
            dimension_semantics=("parallel", "arbitrary")),
    )(thresholds, y_pred, y_true)

    pp = out[:, 0:10, :].sum(axis=(0, 2))
    tp = out[:, 10:20, :].sum(axis=(0, 2))
    npos = out[:, 20, :].sum()
    precisions = (tp + _EPS) / (pp + _EPS)
    recalls = (tp + _EPS) / (npos + _EPS)
    p = jnp.concatenate([jnp.zeros((1,), jnp.float32), precisions,
                         jnp.ones((1,), jnp.float32)])
    r = jnp.concatenate([jnp.ones((1,), jnp.float32), recalls,
                         jnp.zeros((1,), jnp.float32)])
    area = 0.5 * jnp.sum((r[1:] - r[:-1]) * (p[1:] + p[:-1]))
    return jnp.abs(area).astype(jnp.float32)


# speedup vs baseline: 2629.2256x; 1.0125x over previous
import jax
import jax.numpy as jnp
from jax.experimental import pallas as pl
from jax.experimental.pallas import tpu as pltpu

_T = 10
_EPS = 1e-7
_D = 96
_BLK_D = 8
_NB = _D // _BLK_D
_CHUNKS = _BLK_D * 32
_K = 4096.0


def _count_kernel(th_ref, yp_ref, yt_ref, out_ref):
    i = pl.program_id(1)

    @pl.when(i == 0)
    def _init():
        out_ref[...] = jnp.zeros_like(out_ref)

    ts = [th_ref[j] for j in range(_T)]

    def body(c, accs):
        acc_w, acc_t = accs
        d = jax.lax.shift_right_logical(c, 5)
        r = pl.multiple_of(jnp.bitwise_and(c, 31) * 8, 8)
        sc = yp_ref[0, 0, d, pl.ds(r, 8), :]
        yc = yt_ref[0, d, pl.ds(r, 8), :]
        w = jnp.where(yc > 0, 1.0 + _K, 1.0)
        acc_w = acc_w + w
        acc_t = tuple(a + jnp.where(sc > t, w, 0.0)
                      for a, t in zip(acc_t, ts))
        return (acc_w, acc_t)

    z = jnp.zeros((8, 256), jnp.float32)
    acc_w, acc_t = jax.lax.fori_loop(
        0, _CHUNKS, body, (z, (z,) * _T), unroll=4)

    npos = (jnp.sum(acc_w, axis=0) - float(_CHUNKS) * 8.0) * (1.0 / _K)
    out_ref[0, 20:21, :] += npos[None, :]
    for j in range(_T):
        v = jnp.sum(acc_t[j], axis=0)
        tp = jnp.floor(v * (1.0 / _K))
        pp = v - _K * tp
        out_ref[0, j:j + 1, :] += pp[None, :]
        out_ref[0, 10 + j:11 + j, :] += tp[None, :]


def kernel(y_pred, y_true, thresholds):
    out = pl.pallas_call(
        _count_kernel,
        grid=(2, _NB),
        in_specs=[
            pl.BlockSpec(memory_space=pltpu.SMEM),
            pl.BlockSpec((1, 1, _BLK_D, 256, 256),
                         lambda b, i: (b, 1, i, 0, 0)),
            pl.BlockSpec((1, _BLK_D, 256, 256),
                         lambda b, i: (b, i, 0, 0)),
        ],
        out_specs=pl.BlockSpec((1, 24, 256), lambda b, i: (b, 0, 0)),
        out_shape=jax.ShapeDtypeStruct((2, 24, 256), jnp.float32),
        compiler_params=pltpu.CompilerParams(
            dimension_semantics=("parallel", "arbitrary")),
    )(thresholds, y_pred, y_true)

    pp = out[:, 0:10, :].sum(axis=(0, 2))
    tp = out[:, 10:20, :].sum(axis=(0, 2))
    npos = out[:, 20, :].sum()
    precisions = (tp + _EPS) / (pp + _EPS)
    recalls = (tp + _EPS) / (npos + _EPS)
    p = jnp.concatenate([jnp.zeros((1,), jnp.float32), precisions,
                         jnp.ones((1,), jnp.float32)])
    r = jnp.concatenate([jnp.ones((1,), jnp.float32), recalls,
                         jnp.zeros((1,), jnp.float32)])
    area = 0.5 * jnp.sum((r[1:] - r[:-1]) * (p[1:] + p[:-1]))
    return jnp.abs(area).astype(jnp.float32)


# speedup vs baseline: 2808.3549x; 1.0681x over previous
import jax
import jax.numpy as jnp
from jax.experimental import pallas as pl
from jax.experimental.pallas import tpu as pltpu

_T = 10
_EPS = 1e-7
_D = 96
_BLK_D = 32
_NB = _D // _BLK_D
_CHUNKS = _BLK_D * 32
_K = 4096.0


def _count_kernel(th_ref, yp_ref, yt_ref, out_ref):
    i = pl.program_id(1)

    @pl.when(i == 0)
    def _init():
        out_ref[...] = jnp.zeros_like(out_ref)

    ts = [th_ref[j] for j in range(_T)]

    def body(c, accs):
        acc_w, acc_t = accs
        d = jax.lax.shift_right_logical(c, 5)
        r = pl.multiple_of(jnp.bitwise_and(c, 31) * 8, 8)
        sc = yp_ref[0, 0, d, pl.ds(r, 8), :]
        yc = yt_ref[0, d, pl.ds(r, 8), :]
        w = jnp.where(yc > 0, 1.0 + _K, 1.0)
        acc_w = acc_w + w
        acc_t = tuple(a + jnp.where(sc > t, w, 0.0)
                      for a, t in zip(acc_t, ts))
        return (acc_w, acc_t)

    z = jnp.zeros((8, 256), jnp.float32)
    acc_w, acc_t = jax.lax.fori_loop(
        0, _CHUNKS, body, (z, (z,) * _T), unroll=8)

    npos8 = (acc_w - float(_CHUNKS)) * (1.0 / _K)
    out_ref[0, 20:21, :] += jnp.sum(npos8, axis=0)[None, :]
    for j in range(_T):
        v = acc_t[j]
        tp8 = jnp.floor(v * (1.0 / _K))
        pp8 = v - _K * tp8
        out_ref[0, j:j + 1, :] += jnp.sum(pp8, axis=0)[None, :]
        out_ref[0, 10 + j:11 + j, :] += jnp.sum(tp8, axis=0)[None, :]


def kernel(y_pred, y_true, thresholds):
    out = pl.pallas_call(
        _count_kernel,
        grid=(2, _NB),
        in_specs=[
            pl.BlockSpec(memory_space=pltpu.SMEM),
            pl.BlockSpec((1, 1, _BLK_D, 256, 256),
                         lambda b, i: (b, 1, i, 0, 0)),
            pl.BlockSpec((1, _BLK_D, 256, 256),
                         lambda b, i: (b, i, 0, 0)),
        ],
        out_specs=pl.BlockSpec((1, 24, 256), lambda b, i: (b, 0, 0)),
        out_shape=jax.ShapeDtypeStruct((2, 24, 256), jnp.float32),
        compiler_params=pltpu.CompilerParams(
            dimension_semantics=("parallel", "arbitrary")),
    )(thresholds, y_pred, y_true)

    pp = out[:, 0:10, :].sum(axis=(0, 2))
    tp = out[:, 10:20, :].sum(axis=(0, 2))
    npos = out[:, 20, :].sum()
    precisions = (tp + _EPS) / (pp + _EPS)
    recalls = (tp + _EPS) / (npos + _EPS)
    p = jnp.concatenate([jnp.zeros((1,), jnp.float32), precisions,
                         jnp.ones((1,), jnp.float32)])
    r = jnp.concatenate([jnp.ones((1,), jnp.float32), recalls,
                         jnp.zeros((1,), jnp.float32)])
    area = 0.5 * jnp.sum((r[1:] - r[:-1]) * (p[1:] + p[:-1]))
    return jnp.abs(area).astype(jnp.float32)


# speedup vs baseline: 2863.1658x; 1.0195x over previous
import jax
import jax.numpy as jnp
from jax.experimental import pallas as pl
from jax.experimental.pallas import tpu as pltpu

_T = 10
_EPS = 1e-7
_ROWS = 96 * 256
_BLK_R = 32 * 256
_NB = _ROWS // _BLK_R
_CHUNKS = _BLK_R // 8
_K = 4096.0


def _count_kernel(th_ref, yp_ref, yt_ref, out_ref):
    i = pl.program_id(1)

    @pl.when(i == 0)
    def _init():
        out_ref[...] = jnp.zeros_like(out_ref)

    ts = [th_ref[j] for j in range(_T)]

    def body(c, accs):
        acc_w, acc_t = accs
        r = pl.multiple_of(c * 8, 8)
        sc = yp_ref[0, 0, pl.ds(r, 8), :]
        yc = yt_ref[0, pl.ds(r, 8), :]
        w = jnp.where(yc > 0, 1.0 + _K, 1.0)
        acc_w = acc_w + w
        acc_t = tuple(a + jnp.where(sc > t, w, 0.0)
                      for a, t in zip(acc_t, ts))
        return (acc_w, acc_t)

    z = jnp.zeros((8, 256), jnp.float32)
    acc_w, acc_t = jax.lax.fori_loop(
        0, _CHUNKS, body, (z, (z,) * _T), unroll=8)

    npos8 = (acc_w - float(_CHUNKS)) * (1.0 / _K)
    out_ref[0, 20:21, :] += jnp.sum(npos8, axis=0)[None, :]
    for j in range(_T):
        v = acc_t[j]
        tp8 = jnp.floor(v * (1.0 / _K))
        pp8 = v - _K * tp8
        out_ref[0, j:j + 1, :] += jnp.sum(pp8, axis=0)[None, :]
        out_ref[0, 10 + j:11 + j, :] += jnp.sum(tp8, axis=0)[None, :]


def kernel(y_pred, y_true, thresholds):
    yp = y_pred.reshape(2, 2, _ROWS, 256)
    yt = y_true.reshape(2, _ROWS, 256)

    out = pl.pallas_call(
        _count_kernel,
        grid=(2, _NB),
        in_specs=[
            pl.BlockSpec(memory_space=pltpu.SMEM),
            pl.BlockSpec((1, 1, _BLK_R, 256), lambda b, i: (b, 1, i, 0)),
            pl.BlockSpec((1, _BLK_R, 256), lambda b, i: (b, i, 0)),
        ],
        out_specs=pl.BlockSpec((1, 24, 256), lambda b, i: (b, 0, 0)),
        out_shape=jax.ShapeDtypeStruct((2, 24, 256), jnp.float32),
        compiler_params=pltpu.CompilerParams(
            dimension_semantics=("parallel", "arbitrary")),
    )(thresholds, yp, yt)

    pp = out[:, 0:10, :].sum(axis=(0, 2))
    tp = out[:, 10:20, :].sum(axis=(0, 2))
    npos = out[:, 20, :].sum()
    precisions = (tp + _EPS) / (pp + _EPS)
    recalls = (tp + _EPS) / (npos + _EPS)
    p = jnp.concatenate([jnp.zeros((1,), jnp.float32), precisions,
                         jnp.ones((1,), jnp.float32)])
    r = jnp.concatenate([jnp.ones((1,), jnp.float32), recalls,
                         jnp.zeros((1,), jnp.float32)])
    area = 0.5 * jnp.sum((r[1:] - r[:-1]) * (p[1:] + p[:-1]))
    return jnp.abs(area).astype(jnp.float32)


# speedup vs baseline: 2999.7125x; 1.0477x over previous
import jax
import jax.numpy as jnp
from jax.experimental import pallas as pl
from jax.experimental.pallas import tpu as pltpu

_T = 10
_EPS = 1e-7
_ROWS = 96 * 256
_BLK_R = 32 * 256
_NB = _ROWS // _BLK_R
_CHUNKS = _BLK_R // 8
_K = 4096.0


def _count_kernel(th_ref, yp_ref, yt_ref, out_ref):
    i = pl.program_id(1)

    @pl.when(i == 0)
    def _init():
        out_ref[...] = jnp.zeros_like(out_ref)

    ts = [th_ref[j] for j in range(_T)]

    def body(c, accs):
        acc_w, acc_t = accs
        r = pl.multiple_of(c * 8, 8)
        sc = yp_ref[0, 0, pl.ds(r, 8), :]
        yc = yt_ref[0, pl.ds(r, 8), :]
        w = jnp.where(yc > 0, 1.0 + _K, 1.0)
        acc_w = acc_w + w
        acc_t = tuple(a + jnp.where(sc > t, w, 0.0)
                      for a, t in zip(acc_t, ts))
        return (acc_w, acc_t)

    z = jnp.zeros((8, 256), jnp.float32)
    acc_w, acc_t = jax.lax.fori_loop(
        0, _CHUNKS, body, (z, (z,) * _T), unroll=16)

    npos8 = (acc_w - float(_CHUNKS)) * (1.0 / _K)
    out_ref[0, 20:21, :] += jnp.sum(npos8, axis=0)[None, :]
    for j in range(_T):
        v = acc_t[j]
        tp8 = jnp.floor(v * (1.0 / _K))
        pp8 = v - _K * tp8
        out_ref[0, j:j + 1, :] += jnp.sum(pp8, axis=0)[None, :]
        out_ref[0, 10 + j:11 + j, :] += jnp.sum(tp8, axis=0)[None, :]


def kernel(y_pred, y_true, thresholds):
    yp = y_pred.reshape(2, 2, _ROWS, 256)
    yt = y_true.reshape(2, _ROWS, 256)

    out = pl.pallas_call(
        _count_kernel,
        grid=(2, _NB),
        in_specs=[
            pl.BlockSpec(memory_space=pltpu.SMEM),
            pl.BlockSpec((1, 1, _BLK_R, 256), lambda b, i: (b, 1, i, 0)),
            pl.BlockSpec((1, _BLK_R, 256), lambda b, i: (b, i, 0)),
        ],
        out_specs=pl.BlockSpec((1, 24, 256), lambda b, i: (b, 0, 0)),
        out_shape=jax.ShapeDtypeStruct((2, 24, 256), jnp.float32),
        compiler_params=pltpu.CompilerParams(
            dimension_semantics=("parallel", "arbitrary")),
    )(thresholds, yp, yt)

    pp = out[:, 0:10, :].sum(axis=(0, 2))
    tp = out[:, 10:20, :].sum(axis=(0, 2))
    npos = out[:, 20, :].sum()
    precisions = (tp + _EPS) / (pp + _EPS)
    recalls = (tp + _EPS) / (npos + _EPS)
    p = jnp.concatenate([jnp.zeros((1,), jnp.float32), precisions,
                         jnp.ones((1,), jnp.float32)])
    r = jnp.concatenate([jnp.ones((1,), jnp.float32), recalls,
                         jnp.zeros((1,), jnp.float32)])
    area = 0.5 * jnp.sum((r[1:] - r[:-1]) * (p[1:] + p[:-1]))
    return jnp.abs(area).astype(jnp.float32)
